# baseline (speedup 1.0000x reference)
import jax
import jax.numpy as jnp
from jax.experimental import pallas as pl
from jax.experimental.pallas import tpu as pltpu

_MAX_LEN = 200
_EMBED_DIM = 64
_FLAT = _MAX_LEN * _EMBED_DIM
_BB = 256
_NOUT = 16


def _body(pe_ref, *refs):
    out_refs = refs[:_NOUT]
    scratch, sems = refs[_NOUT], refs[_NOUT + 1]
    scratch[...] = jnp.broadcast_to(pe_ref[...], scratch.shape)
    copies = [
        pltpu.make_async_copy(scratch, out_refs[i], sems.at[i])
        for i in range(_NOUT)
    ]
    for c in copies:
        c.start()
    for c in copies:
        c.wait()


def kernel(x, pe_weight):
    pe_flat = pe_weight.reshape(1, _FLAT)
    outs = pl.pallas_call(
        _body,
        in_specs=[pl.BlockSpec(memory_space=pltpu.MemorySpace.VMEM)],
        out_specs=[pl.BlockSpec(memory_space=pltpu.MemorySpace.HBM)] * _NOUT,
        out_shape=[jax.ShapeDtypeStruct((_BB, _FLAT), jnp.float32)] * _NOUT,
        scratch_shapes=[
            pltpu.VMEM((_BB, _FLAT), jnp.float32),
            pltpu.SemaphoreType.DMA((_NOUT,)),
        ],
    )(pe_flat)
    out = jnp.concatenate(outs, axis=0)
    return out.reshape(x.shape[0], _MAX_LEN, _EMBED_DIM)
